# Initial kernel scaffold; baseline (speedup 1.0000x reference)
import jax
import jax.numpy as jnp
from jax.experimental import pallas as pl


def kernel(x, Q, W1, b1, W2, b2, W3, b3):
    raise NotImplementedError("write your pallas kernel here")


# speedup vs baseline: 1.0147x; 1.0147x over previous
import jax
import jax.numpy as jnp
from jax.experimental import pallas as pl
from jax.experimental.pallas import tpu as pltpu

_BLOCK_N = 64


def _fused_kernel(x_ref, q_ref, w1_ref, b1_ref, w2_ref, b2_ref,
                  w3_ref, b3_ref, o_ref):
    nb, d = x_ref.shape

    xp = jnp.dot(x_ref[...], q_ref[...], preferred_element_type=jnp.float32)

    i_idx = jax.lax.broadcasted_iota(jnp.int32, (d, d), 0)
    j_idx = jax.lax.broadcasted_iota(jnp.int32, (d, d), 1)
    tri = (j_idx < i_idx).astype(jnp.float32)
    xm = (xp[:, None, :] * tri[None, :, :]).reshape(nb * d, d)
    xm = xm.astype(jnp.bfloat16)

    h = jnp.dot(xm, w1_ref[...], preferred_element_type=jnp.float32)
    h = jnp.maximum(h + b1_ref[...], 0.0).astype(jnp.bfloat16)
    h = jnp.dot(h, w2_ref[...], preferred_element_type=jnp.float32)
    h = jnp.maximum(h + b2_ref[...], 0.0).astype(jnp.bfloat16)
    o_ref[...] = (
        jnp.dot(h, w3_ref[...], preferred_element_type=jnp.float32)
        + b3_ref[...])


@jax.jit
def _forward(x, Q, W1, b1, W2, b2, W3, b3):
    n, d = x.shape
    n_params = W3.shape[1]
    nb = _BLOCK_N

    xb = x.astype(jnp.bfloat16)
    qb = Q.astype(jnp.bfloat16)
    w1b = W1.astype(jnp.bfloat16)
    w2b = W2.astype(jnp.bfloat16)
    w3b = W3.astype(jnp.bfloat16)

    const = lambda i: (0, 0)

    out = pl.pallas_call(
        _fused_kernel,
        grid=(n // nb,),
        in_specs=[
            pl.BlockSpec((nb, d), lambda i: (i, 0)),
            pl.BlockSpec(qb.shape, const),
            pl.BlockSpec(w1b.shape, const),
            pl.BlockSpec(b1.shape, const),
            pl.BlockSpec(w2b.shape, const),
            pl.BlockSpec(b2.shape, const),
            pl.BlockSpec(w3b.shape, const),
            pl.BlockSpec(b3.shape, const),
        ],
        out_specs=pl.BlockSpec((nb * d, n_params), lambda i: (i, 0)),
        out_shape=jax.ShapeDtypeStruct((n * d, n_params), jnp.float32),
        compiler_params=pltpu.CompilerParams(
            dimension_semantics=("parallel",)),
    )(xb, qb, w1b, b1, w2b, b2, w3b, b3)

    return out.reshape(n, n_params * d)


def kernel(x, Q, W1, b1, W2, b2, W3, b3):
    return _forward(x, Q, W1, b1, W2, b2, W3, b3)
